# Initial kernel scaffold; baseline (speedup 1.0000x reference)
import jax
import jax.numpy as jnp
from jax.experimental import pallas as pl


def kernel(indices, offsets, table):
    raise NotImplementedError("write your pallas kernel here")


# speedup vs baseline: 188.1355x; 188.1355x over previous
import functools

import jax
import jax.numpy as jnp
from jax import lax
from jax.experimental import pallas as pl
from jax.experimental.pallas import tpu as pltpu
from jax.experimental.pallas import tpu_sc as plsc

NC = 2
NS = 16
L = 16
K = 1280
NSTR = K // 128
INT_MAX = 2**31 - 1


def _embedding_bag_sc(indices, offsets, table):
  total = indices.shape[0]
  nbags = offsets.shape[0] - 1
  vocab, d = table.shape
  nw = NC * NS
  c_per_w = total // nw
  g_per_w = c_per_w // K
  rows_per_w = nbags // NS
  noff = nbags + 1
  noff_pad = ((noff + L - 1) // L + 1) * L

  mesh = plsc.VectorSubcoreMesh(core_axis_name="c", subcore_axis_name="s")

  @functools.partial(
      pl.kernel,
      out_type=jax.ShapeDtypeStruct((NC, nbags, d), jnp.float32),
      mesh=mesh,
      compiler_params=pltpu.CompilerParams(needs_layout_passes=False,
                                           use_tc_tiling_on_sc=False),
      scratch_types=dict(
          offs_v=pltpu.VMEM((noff_pad,), jnp.int32),
          idx2d=pltpu.VMEM((NSTR, 128), jnp.int32),
          seg2d=pltpu.VMEM((NSTR, 128), jnp.int32),
          m_v=pltpu.VMEM((K,), jnp.int32),
          rows_v=pltpu.VMEM((K, d), jnp.float32),
          z_v=pltpu.VMEM((128, d), jnp.float32),
          acc_sh=pltpu.VMEM_SHARED((nbags, d), jnp.float32),
          isem=pltpu.SemaphoreType.DMA,
          gsem=pltpu.SemaphoreType.DMA,
      ),
  )
  def emb_kernel(idx_hbm, off_hbm, table_hbm, out_hbm, *, offs_v, idx2d,
                 seg2d, m_v, rows_v, z_v, acc_sh, isem, gsem):
    c = lax.axis_index("c")
    s = lax.axis_index("s")
    wid = c * NS + s
    p0 = wid * c_per_w

    pltpu.sync_copy(off_hbm, offs_v.at[pl.ds(0, noff)])
    for i in range(noff - 1, noff_pad, L):
      base = (i // L) * L
      keep = lax.iota(jnp.int32, L) + base < noff - 1
      prev = offs_v[pl.ds(base, L)]
      offs_v[pl.ds(base, L)] = jnp.where(keep, prev, INT_MAX)

    zeros16 = jnp.zeros((L,), jnp.float32)
    for r in range(128):
      for q in range(d // L):
        z_v[r, pl.ds(q * L, L)] = zeros16
    for k in range(rows_per_w // 128):
      pltpu.sync_copy(z_v, acc_sh.at[pl.ds(s * rows_per_w + k * 128, 128)])
    plsc.subcore_barrier()

    def val_at(i):
      v = offs_v[pl.ds(i, L)]
      return v[0]

    def lower_bound(target):
      def bs_body(_, lohi):
        lo, hi = lohi
        mid = (lo + hi) // 2
        ge = val_at(mid) >= target
        return (jnp.where(ge, lo, mid + 1), jnp.where(ge, mid, hi))
      lo, _ = lax.fori_loop(0, 15, bs_body,
                            (jnp.int32(0), jnp.int32(noff - 1)))
      return lo

    lo0 = lower_bound(p0)
    carry0 = lo0 - 1

    iota16 = lax.iota(jnp.int32, L)
    neg1 = jnp.full((L,), -1, jnp.int32)

    def chunk_body(g, state):
      lo_c, carry = state
      p = p0 + g * K

      icps = [
          pltpu.async_copy(idx_hbm.at[pl.ds(p + j * 128, 128)],
                           idx2d.at[j], isem)
          for j in range(NSTR)
      ]
      for cp in icps:
        cp.wait()
      gcps = [
          pltpu.async_copy(table_hbm.at[idx2d.at[j]],
                           rows_v.at[pl.ds(j * 128, 128)], gsem)
          for j in range(NSTR)
      ]

      for i in range(K // L):
        m_v[pl.ds(i * L, L)] = neg1

      hi = lower_bound(p + K)
      bo_al = (lo_c // L) * L
      nvec = jnp.where(hi > lo_c, (hi - 1) // L - lo_c // L + 1, 0)

      def off_body(t, _):
        bo = bo_al + t * L
        ovec = offs_v[pl.ds(bo, L)]
        nxt = offs_v[pl.ds(bo + 1, L)]
        in_range = (ovec >= p) & (ovec < p + K)
        is_last = ovec != nxt
        bvec = bo + iota16
        plsc.store_scatter(m_v, [ovec - p], bvec, mask=is_last & in_range)
        return 0

      lax.fori_loop(0, nvec, off_body, 0)

      for i in range(K // L):
        mv = m_v[pl.ds(i * L, L)]
        seg_vec = jnp.maximum(plsc.cummax(mv), carry)
        seg2d[i // 8, pl.ds((i % 8) * L, L)] = seg_vec
        carry = seg_vec[L - 1]

      for cp in gcps:
        cp.wait()
      for j in range(NSTR):
        pltpu.sync_copy(rows_v.at[pl.ds(j * 128, 128)],
                        acc_sh.at[seg2d.at[j]], add=True)
      return (hi, carry)

    lax.fori_loop(0, g_per_w, chunk_body, (lo0, carry0))
    plsc.subcore_barrier()

    for k in range(rows_per_w // 1024):
      sl = pl.ds(s * rows_per_w + k * 1024, 1024)
      pltpu.sync_copy(acc_sh.at[sl], rows_v.at[pl.ds(0, 1024)])
      pltpu.sync_copy(rows_v.at[pl.ds(0, 1024)], out_hbm.at[c, sl])

  return emb_kernel(indices, offsets, table)


def _combine_body(p_ref, o_ref):
  o_ref[...] = p_ref[0] + p_ref[1]


def _combine(partial):
  nc, n = partial.shape
  p3 = partial.reshape(nc, n // 128, 128)
  grid = 8
  rows = (n // 128) // grid
  out = pl.pallas_call(
      _combine_body,
      out_shape=jax.ShapeDtypeStruct((n // 128, 128), jnp.float32),
      grid=(grid,),
      in_specs=[pl.BlockSpec((nc, rows, 128), lambda i: (0, i, 0))],
      out_specs=pl.BlockSpec((rows, 128), lambda i: (i, 0)),
  )(p3)
  return out.reshape(n)


def kernel(indices, offsets, table):
  nbags = offsets.shape[0] - 1
  d = table.shape[1]
  partial = _embedding_bag_sc(indices, offsets, table)
  out = _combine(partial.reshape(NC, nbags * d))
  return out.reshape(nbags, d)


# speedup vs baseline: 198.9846x; 1.0577x over previous
import functools

import jax
import jax.numpy as jnp
from jax import lax
from jax.experimental import pallas as pl
from jax.experimental.pallas import tpu as pltpu
from jax.experimental.pallas import tpu_sc as plsc

NC = 2
NS = 16
L = 16
K = 640
NSTR = K // 128
INT_MAX = 2**31 - 1


def _embedding_bag_sc(indices, offsets, table):
  total = indices.shape[0]
  nbags = offsets.shape[0] - 1
  vocab, d = table.shape
  nw = NC * NS
  c_per_w = total // nw
  g_per_w = c_per_w // K
  rows_per_w = nbags // NS
  noff = nbags + 1
  noff_pad = ((noff + L - 1) // L + 1) * L

  mesh = plsc.VectorSubcoreMesh(core_axis_name="c", subcore_axis_name="s")

  @functools.partial(
      pl.kernel,
      out_type=jax.ShapeDtypeStruct((NC, nbags, d), jnp.float32),
      mesh=mesh,
      compiler_params=pltpu.CompilerParams(needs_layout_passes=False,
                                           use_tc_tiling_on_sc=False),
      scratch_types=dict(
          offs_v=pltpu.VMEM((noff_pad,), jnp.int32),
          idx_a=pltpu.VMEM((NSTR, 128), jnp.int32),
          idx_b=pltpu.VMEM((NSTR, 128), jnp.int32),
          seg_a=pltpu.VMEM((NSTR, 128), jnp.int32),
          seg_b=pltpu.VMEM((NSTR, 128), jnp.int32),
          m_v=pltpu.VMEM((K,), jnp.int32),
          rows_a=pltpu.VMEM((K, d), jnp.float32),
          rows_b=pltpu.VMEM((K, d), jnp.float32),
          acc_sh=pltpu.VMEM_SHARED((nbags, d), jnp.float32),
          isem=pltpu.SemaphoreType.DMA,
          gsem=pltpu.SemaphoreType.DMA,
          ssem_a=pltpu.SemaphoreType.DMA,
          ssem_b=pltpu.SemaphoreType.DMA,
      ),
  )
  def emb_kernel(idx_hbm, off_hbm, table_hbm, out_hbm, *, offs_v, idx_a,
                 idx_b, seg_a, seg_b, m_v, rows_a, rows_b, acc_sh, isem,
                 gsem, ssem_a, ssem_b):
    c = lax.axis_index("c")
    s = lax.axis_index("s")
    wid = c * NS + s
    p0 = wid * c_per_w

    pltpu.sync_copy(off_hbm, offs_v.at[pl.ds(0, noff)])
    for i in range(noff - 1, noff_pad, L):
      base = (i // L) * L
      keep = lax.iota(jnp.int32, L) + base < noff - 1
      prev = offs_v[pl.ds(base, L)]
      offs_v[pl.ds(base, L)] = jnp.where(keep, prev, INT_MAX)

    zeros16 = jnp.zeros((L,), jnp.float32)
    for r in range(128):
      for q in range(d // L):
        rows_a[r, pl.ds(q * L, L)] = zeros16
    for k in range(rows_per_w // 128):
      pltpu.sync_copy(rows_a.at[pl.ds(0, 128)],
                      acc_sh.at[pl.ds(s * rows_per_w + k * 128, 128)])
    plsc.subcore_barrier()

    def val_at(i):
      v = offs_v[pl.ds(i, L)]
      return v[0]

    def lower_bound(target):
      def bs_body(_, lohi):
        lo, hi = lohi
        mid = (lo + hi) // 2
        ge = val_at(mid) >= target
        return (jnp.where(ge, lo, mid + 1), jnp.where(ge, mid, hi))
      lo, _ = lax.fori_loop(0, 15, bs_body,
                            (jnp.int32(0), jnp.int32(noff - 1)))
      return lo

    lo0 = lower_bound(p0)
    carry0 = lo0 - 1

    iota16 = lax.iota(jnp.int32, L)
    neg1 = jnp.full((L,), -1, jnp.int32)
    bufs = ((idx_a, seg_a, rows_a, ssem_a), (idx_b, seg_b, rows_b, ssem_b))

    def fire_idx(g, idx2d):
      p = p0 + g * K
      for j in range(NSTR):
        pltpu.async_copy(idx_hbm.at[pl.ds(p + j * 128, 128)],
                         idx2d.at[j], isem)

    def drain_idx(g, idx2d):
      p = p0 + g * K
      for j in range(NSTR):
        pltpu.make_async_copy(idx_hbm.at[pl.ds(p + j * 128, 128)],
                              idx2d.at[j], isem).wait()

    def compute_seg(g, lo_c, carry, seg2d):
      p = p0 + g * K
      for i in range(K // L):
        m_v[pl.ds(i * L, L)] = neg1

      hi = lower_bound(p + K)
      bo_al = (lo_c // L) * L
      nvec = jnp.where(hi > lo_c, (hi - 1) // L - lo_c // L + 1, 0)

      def off_body(t, _):
        bo = bo_al + t * L
        ovec = offs_v[pl.ds(bo, L)]
        nxt = offs_v[pl.ds(bo + 1, L)]
        in_range = (ovec >= p) & (ovec < p + K)
        is_last = ovec != nxt
        bvec = bo + iota16
        plsc.store_scatter(m_v, [ovec - p], bvec, mask=is_last & in_range)
        return 0

      lax.fori_loop(0, nvec, off_body, 0)

      for i in range(K // L):
        mv = m_v[pl.ds(i * L, L)]
        seg_vec = jnp.maximum(plsc.cummax(mv), carry)
        seg2d[i // 8, pl.ds((i % 8) * L, L)] = seg_vec
        carry = seg_vec[L - 1]
      return hi, carry

    fire_idx(0, idx_a)

    def pair_body(gg, state):
      lo_c, carry = state
      for par in range(2):
        g = gg + par
        idx2d, seg2d, rows_v, ssem = bufs[par]

        @pl.when(g >= 2)
        def _():
          pltpu.make_async_copy(table_hbm.at[pl.ds(0, K)], rows_v,
                                ssem).wait()

        drain_idx(g, idx2d)
        gcps = [
            pltpu.async_copy(table_hbm.at[idx2d.at[j]],
                             rows_v.at[pl.ds(j * 128, 128)], gsem)
            for j in range(NSTR)
        ]

        lo_c, carry = compute_seg(g, lo_c, carry, seg2d)

        @pl.when(g + 1 < g_per_w)
        def _():
          fire_idx(g + 1, bufs[1 - par][0])

        for cp in gcps:
          cp.wait()
        for j in range(NSTR):
          pltpu.async_copy(rows_v.at[pl.ds(j * 128, 128)],
                           acc_sh.at[seg2d.at[j]], ssem, add=True)
      return (lo_c, carry)

    lax.fori_loop(0, g_per_w // 2, lambda t, st: pair_body(2 * t, st),
                  (lo0, carry0))
    pltpu.make_async_copy(table_hbm.at[pl.ds(0, K)], rows_a, ssem_a).wait()
    pltpu.make_async_copy(table_hbm.at[pl.ds(0, K)], rows_b, ssem_b).wait()
    plsc.subcore_barrier()

    for k in range(rows_per_w // 512):
      sl = pl.ds(s * rows_per_w + k * 512, 512)
      pltpu.sync_copy(acc_sh.at[sl], rows_a.at[pl.ds(0, 512)])
      pltpu.sync_copy(rows_a.at[pl.ds(0, 512)], out_hbm.at[c, sl])

  return emb_kernel(indices, offsets, table)


def _combine_body(p_ref, o_ref):
  o_ref[...] = p_ref[0] + p_ref[1]


def _combine(partial):
  nc, n = partial.shape
  p3 = partial.reshape(nc, n // 128, 128)
  grid = 8
  rows = (n // 128) // grid
  out = pl.pallas_call(
      _combine_body,
      out_shape=jax.ShapeDtypeStruct((n // 128, 128), jnp.float32),
      grid=(grid,),
      in_specs=[pl.BlockSpec((nc, rows, 128), lambda i: (0, i, 0))],
      out_specs=pl.BlockSpec((rows, 128), lambda i: (i, 0)),
  )(p3)
  return out.reshape(n)


def kernel(indices, offsets, table):
  nbags = offsets.shape[0] - 1
  d = table.shape[1]
  partial = _embedding_bag_sc(indices, offsets, table)
  out = _combine(partial.reshape(NC, nbags * d))
  return out.reshape(nbags, d)
